# Initial kernel scaffold; baseline (speedup 1.0000x reference)
import jax
import jax.numpy as jnp
from jax.experimental import pallas as pl


def kernel(xyz, features, seed_xyz, W1, g1, b1, m1, v1, W2, g2, b2, m2, v2, W3, g3, b3, m3, v3):
    raise NotImplementedError("write your pallas kernel here")


# speedup vs baseline: 8.5698x; 8.5698x over previous
import functools

import jax
import jax.numpy as jnp
import numpy as np
from jax import lax
from jax.experimental import pallas as pl
from jax.experimental.pallas import tpu as pltpu
from jax.experimental.pallas import tpu_sc as plsc

RADIUS = 0.3
NSAMPLE = 16
NPOINT = 512
EPS = 1e-5

B = 4
N = 16384
C = 256
SUB = 8
NL = N // SUB
D_ROW = 128
C_TILE = 32
M_TILE = 64


def _rmax(x):
    return jnp.max(jnp.max(x, axis=1, keepdims=True), axis=0, keepdims=True)


def _rmin(x):
    return jnp.min(jnp.min(x, axis=1, keepdims=True), axis=0, keepdims=True)


def _rsum(x):
    return jnp.sum(jnp.sum(x, axis=1, keepdims=True), axis=0, keepdims=True)


def _fps_body(seed_ref, xyz_ref, inds_ref, nxyz_ref):
    sx = seed_ref[0, 0]
    sy = seed_ref[0, 1]
    sz = seed_ref[0, 2]
    xx = xyz_ref[0, 0]
    xy = xyz_ref[0, 1]
    xz = xyz_ref[0, 2]
    lin = (lax.broadcasted_iota(jnp.int32, (SUB, NL), 0) * NL
           + lax.broadcasted_iota(jnp.int32, (SUB, NL), 1))
    lin512 = (lax.broadcasted_iota(jnp.int32, (SUB, 64), 0) * 64
              + lax.broadcasted_iota(jnp.int32, (SUB, 64), 1))

    def body(i, state):
        dist, f, inds, nx, ny, nz = state
        mask = lin == f
        onehot = lin512 == i
        inds = jnp.where(onehot, f, inds)
        cx = _rsum(jnp.where(mask, sx, 0.0))
        cy = _rsum(jnp.where(mask, sy, 0.0))
        cz = _rsum(jnp.where(mask, sz, 0.0))
        nx = jnp.where(onehot, _rsum(jnp.where(mask, xx, 0.0)), nx)
        ny = jnp.where(onehot, _rsum(jnp.where(mask, xy, 0.0)), ny)
        nz = jnp.where(onehot, _rsum(jnp.where(mask, xz, 0.0)), nz)
        dx = sx - cx
        dy = sy - cy
        dz = sz - cz
        d = (dx * dx + dy * dy) + dz * dz
        dist = jnp.minimum(dist, d)
        m = _rmax(dist)
        cand = jnp.where(dist == m, lin, N)
        f_next = _rmin(cand)
        return dist, f_next, inds, nx, ny, nz

    init = (jnp.full((SUB, NL), 1e10, jnp.float32),
            jnp.zeros((1, 1), jnp.int32),
            jnp.zeros((SUB, 64), jnp.int32),
            jnp.zeros((SUB, 64), jnp.float32),
            jnp.zeros((SUB, 64), jnp.float32),
            jnp.zeros((SUB, 64), jnp.float32))
    _, _, inds, nx, ny, nz = lax.fori_loop(0, NPOINT, body, init)
    inds_ref[0] = inds
    nxyz_ref[0, 0] = nx
    nxyz_ref[0, 1] = ny
    nxyz_ref[0, 2] = nz


def _run_fps(seed_t4, xyz_t4):
    return pl.pallas_call(
        _fps_body,
        grid=(B,),
        in_specs=[
            pl.BlockSpec((1, 3, SUB, NL), lambda b: (b, 0, 0, 0)),
            pl.BlockSpec((1, 3, SUB, NL), lambda b: (b, 0, 0, 0)),
        ],
        out_specs=[
            pl.BlockSpec((1, SUB, 64), lambda b: (b, 0, 0)),
            pl.BlockSpec((1, 3, SUB, 64), lambda b: (b, 0, 0, 0)),
        ],
        out_shape=[
            jax.ShapeDtypeStruct((B, SUB, 64), jnp.int32),
            jax.ShapeDtypeStruct((B, 3, SUB, 64), jnp.float32),
        ],
    )(seed_t4, xyz_t4)


def _ballq_body(xyz_ref, ctr_ref, idx_ref):
    b = pl.program_id(0)
    x = xyz_ref[0, 0:1, :]
    y = xyz_ref[0, 1:2, :]
    z = xyz_ref[0, 2:3, :]
    c = ctr_ref[0]
    cx = c[:, 0:1]
    cy = c[:, 1:2]
    cz = c[:, 2:3]
    dx = cx - x
    dy = cy - y
    dz = cz - z
    d2 = (dx * dx + dy * dy) + dz * dz
    lin = lax.broadcasted_iota(jnp.int32, (C_TILE, N), 1)
    scores = jnp.where(d2 < np.float32(RADIUS * RADIUS), lin, N)
    iota16 = lax.broadcasted_iota(jnp.int32, (1, NSAMPLE), 1)
    idxmat = jnp.zeros((C_TILE, NSAMPLE), jnp.int32)
    first = jnp.full((C_TILE, 1), N, jnp.int32)
    for k in range(NSAMPLE):
        m = jnp.min(scores, axis=1, keepdims=True)
        if k == 0:
            first = m
        slotval = jnp.where(m < N, m, first)
        scores = jnp.where(scores == m, N, scores)
        idxmat = jnp.where(iota16 == k, slotval, idxmat)
    idx_ref[0] = idxmat + b * N


def _run_ballq(xyz_t3, centers):
    return pl.pallas_call(
        _ballq_body,
        grid=(B, NPOINT // C_TILE),
        in_specs=[
            pl.BlockSpec((1, 3, N), lambda b, t: (b, 0, 0)),
            pl.BlockSpec((1, C_TILE, 3), lambda b, t: (b, t, 0)),
        ],
        out_specs=pl.BlockSpec((1, C_TILE, NSAMPLE), lambda b, t: (b, t, 0)),
        out_shape=jax.ShapeDtypeStruct((B, NPOINT, NSAMPLE), jnp.int32),
    )(xyz_t3, centers)


def _table_body(f_ref, xyz_ref, wt_ref, wg_ref, out_ref):
    f = f_ref[0]
    wt = wt_ref[...]
    out128 = lax.dot_general(f, wt, (((0,), (0,)), ((), ())),
                             preferred_element_type=jnp.float32)
    xyzb = xyz_ref[0]
    wg = wg_ref[...]
    xyzproj = (xyzb[:, 0:1] * wg[0:1, :] + xyzb[:, 1:2] * wg[1:2, :]
               + xyzb[:, 2:3] * wg[2:3, :])
    out_ref[0] = out128 + xyzproj


def _run_table(features, xyz, w1f_t, wg):
    n_tile = 512
    return pl.pallas_call(
        _table_body,
        grid=(B, N // n_tile),
        in_specs=[
            pl.BlockSpec((1, C, n_tile), lambda b, t: (b, 0, t)),
            pl.BlockSpec((1, n_tile, 3), lambda b, t: (b, t, 0)),
            pl.BlockSpec((C, 128), lambda b, t: (0, 0)),
            pl.BlockSpec((3, 128), lambda b, t: (0, 0)),
        ],
        out_specs=pl.BlockSpec((1, n_tile, D_ROW), lambda b, t: (b, t, 0)),
        out_shape=jax.ShapeDtypeStruct((B, N, D_ROW), jnp.float32),
    )(features, xyz, w1f_t, wg)


_SC_CHUNK = 128


def _sc_gather(table, idx):
    tot = idx.shape[0]
    info = plsc.get_sparse_core_info()
    nw = info.num_cores * info.num_subcores
    per_w = tot // nw
    n_chunk = per_w // _SC_CHUNK
    mesh = plsc.VectorSubcoreMesh(core_axis_name="c", subcore_axis_name="s")

    @functools.partial(
        pl.kernel,
        mesh=mesh,
        out_type=jax.ShapeDtypeStruct((tot, D_ROW), jnp.float32),
        scratch_types=[
            pltpu.VMEM((_SC_CHUNK,), jnp.int32),
            pltpu.VMEM((_SC_CHUNK, D_ROW), jnp.float32),
            pltpu.SemaphoreType.DMA,
        ],
    )
    def k(tab_hbm, idx_hbm, out_hbm, idx_v, rows_v, sem):
        wid = lax.axis_index("s") * info.num_cores + lax.axis_index("c")
        base = wid * per_w

        def body(ci, carry):
            off = base + ci * _SC_CHUNK
            pltpu.sync_copy(idx_hbm.at[pl.ds(off, _SC_CHUNK)], idx_v)
            pltpu.async_copy(tab_hbm.at[idx_v], rows_v, sem).wait()
            pltpu.sync_copy(rows_v, out_hbm.at[pl.ds(off, _SC_CHUNK)])
            return carry

        lax.fori_loop(0, n_chunk, body, 0)

    return k(table, idx)


def _mlp_body(g_ref, ctr_ref, wg_ref, w2_ref, w3_ref,
              o1_ref, o2_ref, o3_ref, out_ref):
    h1p = g_ref[0]
    c = ctr_ref[0]
    wg = wg_ref[...]
    cc = (c[:, 0:1] * wg[0:1, :] + c[:, 1:2] * wg[1:2, :]
          + c[:, 2:3] * wg[2:3, :])
    ccrep = jnp.reshape(
        jnp.broadcast_to(cc[:, None, :], (M_TILE, NSAMPLE, 128)),
        (M_TILE * NSAMPLE, 128))
    h1 = jnp.maximum(h1p - ccrep + o1_ref[...], 0.0)
    h2 = jnp.maximum(
        lax.dot_general(h1, w2_ref[...], (((1,), (0,)), ((), ())),
                        preferred_element_type=jnp.float32) + o2_ref[...],
        0.0)
    h3 = jnp.maximum(
        lax.dot_general(h2, w3_ref[...], (((1,), (0,)), ((), ())),
                        preferred_element_type=jnp.float32) + o3_ref[...],
        0.0)
    out_ref[0] = jnp.max(jnp.reshape(h3, (M_TILE, NSAMPLE, 128)), axis=1)


def _run_mlp(grows, centers, wg, w2t, w3t, o1, o2, o3):
    rows_tile = M_TILE * NSAMPLE
    return pl.pallas_call(
        _mlp_body,
        grid=(B, NPOINT // M_TILE),
        in_specs=[
            pl.BlockSpec((1, rows_tile, D_ROW), lambda b, t: (b, t, 0)),
            pl.BlockSpec((1, M_TILE, 3), lambda b, t: (b, t, 0)),
            pl.BlockSpec((3, 128), lambda b, t: (0, 0)),
            pl.BlockSpec((128, 128), lambda b, t: (0, 0)),
            pl.BlockSpec((128, 128), lambda b, t: (0, 0)),
            pl.BlockSpec((1, 128), lambda b, t: (0, 0)),
            pl.BlockSpec((1, 128), lambda b, t: (0, 0)),
            pl.BlockSpec((1, 128), lambda b, t: (0, 0)),
        ],
        out_specs=pl.BlockSpec((1, M_TILE, 128), lambda b, t: (b, t, 0)),
        out_shape=jax.ShapeDtypeStruct((B, NPOINT, 128), jnp.float32),
    )(grows, centers, wg, w2t, w3t, o1, o2, o3)


def kernel(xyz, features, seed_xyz, W1, g1, b1, m1, v1,
           W2, g2, b2, m2, v2, W3, g3, b3, m3, v3):
    xyz_t3 = jnp.transpose(xyz, (0, 2, 1))
    seed_t4 = jnp.reshape(jnp.transpose(seed_xyz, (0, 2, 1)), (B, 3, SUB, NL))
    xyz_t4 = jnp.reshape(xyz_t3, (B, 3, SUB, NL))

    s1 = g1 / jnp.sqrt(v1 + EPS)
    s2 = g2 / jnp.sqrt(v2 + EPS)
    s3 = g3 / jnp.sqrt(v3 + EPS)
    o1 = (b1 - m1 * s1)[None, :]
    o2 = (b2 - m2 * s2)[None, :]
    o3 = (b3 - m3 * s3)[None, :]
    w1f_t = jnp.transpose(W1[:, 3:] * s1[:, None])
    wg = jnp.transpose(W1[:, :3] * (s1 / RADIUS)[:, None])
    w2t = jnp.transpose(W2 * s2[:, None])
    w3t = jnp.transpose(W3 * s3[:, None])

    inds8, nxyz8 = _run_fps(seed_t4, xyz_t4)
    sample_inds = jnp.reshape(inds8, (B, NPOINT))
    new_xyz = jnp.transpose(jnp.reshape(nxyz8, (B, 3, NPOINT)), (0, 2, 1))

    idx = _run_ballq(xyz_t3, new_xyz)
    table = _run_table(features, xyz, w1f_t, wg)

    grows = _sc_gather(jnp.reshape(table, (B * N, D_ROW)),
                       jnp.reshape(idx, (B * NPOINT * NSAMPLE,)))
    out = _run_mlp(jnp.reshape(grows, (B, NPOINT * NSAMPLE, D_ROW)),
                   new_xyz, wg, w2t, w3t, o1, o2, o3)
    new_features = jnp.transpose(out, (0, 2, 1))
    return (new_xyz, new_features, sample_inds)
